# baseline (speedup 1.0000x reference)
import jax
import jax.numpy as jnp
from jax.experimental import pallas as pl
from jax.experimental.pallas import tpu as pltpu

_BM = 256


def _body(x_ref, s_ref, war_ref, wma_ref, ar_ref, ma_ref, b_ref, o_ref,
          xw_ref, sw_ref):
    i = pl.program_id(0)

    @pl.when(i == 0)
    def _premix():
        xw_ref[...] = jnp.dot(
            x_ref[...].astype(jnp.bfloat16), war_ref[...],
            preferred_element_type=jnp.float32).astype(jnp.bfloat16)
        sw_ref[...] = jnp.dot(
            s_ref[...].astype(jnp.bfloat16), wma_ref[...],
            preferred_element_type=jnp.float32).astype(jnp.bfloat16)

    acc = jnp.dot(ar_ref[...].astype(jnp.bfloat16), xw_ref[...],
                  preferred_element_type=jnp.float32)
    acc = acc + jnp.dot(ma_ref[...].astype(jnp.bfloat16), sw_ref[...],
                        preferred_element_type=jnp.float32)
    o_ref[...] = jnp.maximum(acc + b_ref[...], 0.0)


def kernel(x, arma_conv_AR, arma_conv_MA, input_signal, ar_kernel, ma_kernel, bias):
    n, f_in = x.shape
    f_out = ar_kernel.shape[1]

    war16 = ar_kernel.astype(jnp.bfloat16)
    wma16 = ma_kernel.astype(jnp.bfloat16)

    out = pl.pallas_call(
        _body,
        grid=(n // _BM,),
        in_specs=[
            pl.BlockSpec((n, f_in), lambda i: (0, 0)),
            pl.BlockSpec((n, f_in), lambda i: (0, 0)),
            pl.BlockSpec((f_in, f_out), lambda i: (0, 0)),
            pl.BlockSpec((f_in, f_out), lambda i: (0, 0)),
            pl.BlockSpec((_BM, n), lambda i: (i, 0)),
            pl.BlockSpec((_BM, n), lambda i: (i, 0)),
            pl.BlockSpec((1, f_out), lambda i: (0, 0)),
        ],
        out_specs=pl.BlockSpec((_BM, f_out), lambda i: (i, 0)),
        out_shape=jax.ShapeDtypeStruct((n, f_out), jnp.float32),
        scratch_shapes=[
            pltpu.VMEM((n, f_out), jnp.bfloat16),
            pltpu.VMEM((n, f_out), jnp.bfloat16),
        ],
        compiler_params=pltpu.CompilerParams(
            dimension_semantics=("arbitrary",)),
    )(x, input_signal, war16, wma16, arma_conv_AR, arma_conv_MA,
      bias.reshape(1, f_out))

    return out
